# baseline (speedup 1.0000x reference)
import functools

import jax
import jax.numpy as jnp
from jax import lax
from jax.experimental import pallas as pl
from jax.experimental.pallas import tpu as pltpu
from jax.experimental.pallas import tpu_sc as plsc

_OBS = (100, 100, 100)
_NUM_ACTIONS = 16
_BATCH = 16384

_NC = 2
_NS = 16
_NW = _NC * _NS
_BPW = _BATCH // _NW
_LANES = 16
_GCHUNK = 128
_NGATHER = _BPW // _GCHUNK
_PACK = 128 // _NUM_ACTIONS
_PROWS = 1000000 // _PACK

_TCB = 1024
_TCG = _BATCH // _TCB


@functools.partial(
    pl.kernel,
    out_type=(jax.ShapeDtypeStruct((_BATCH, 128), jnp.float32),
              jax.ShapeDtypeStruct((_BATCH,), jnp.int32)),
    mesh=plsc.VectorSubcoreMesh(core_axis_name="c", subcore_axis_name="s"),
    scratch_types=[
        pltpu.VMEM((_BPW,), jnp.int32),
        pltpu.VMEM((_BPW,), jnp.int32),
        pltpu.VMEM((_BPW,), jnp.int32),
        pltpu.VMEM((_BPW,), jnp.int32),
        pltpu.VMEM((_BPW,), jnp.int32),
        pltpu.VMEM((_BPW, 128), jnp.float32),
        pltpu.SemaphoreType.DMA,
    ],
)
def _sc_gather(x0_hbm, x1_hbm, x2_hbm, tbl_hbm, wide_hbm, col_hbm,
               x0_v, x1_v, x2_v, row_v, col_v, rows_v, sem):
    wid = lax.axis_index("s") * _NC + lax.axis_index("c")
    base = wid * _BPW

    pltpu.sync_copy(x0_hbm.at[pl.ds(base, _BPW)], x0_v)
    pltpu.sync_copy(x1_hbm.at[pl.ds(base, _BPW)], x1_v)
    pltpu.sync_copy(x2_hbm.at[pl.ds(base, _BPW)], x2_v)

    hi = jnp.full((_LANES,), _OBS[0] - 1, jnp.int32)
    lo = jnp.zeros((_LANES,), jnp.int32)
    for i in range(_BPW // _LANES):
        sl = pl.ds(i * _LANES, _LANES)
        a = jnp.minimum(jnp.maximum(x0_v[sl], lo), hi)
        b = jnp.minimum(jnp.maximum(x1_v[sl], lo), hi)
        c = jnp.minimum(jnp.maximum(x2_v[sl], lo), hi)
        idx = (a * (_OBS[1] * _OBS[2]) + b * _OBS[2]) + c
        row_v[sl] = lax.shift_right_logical(idx, 3)
        col_v[sl] = lax.shift_left(jnp.bitwise_and(idx, 7), 4)

    cps = [pltpu.async_copy(col_v, col_hbm.at[pl.ds(base, _BPW)], sem)]
    for j in range(_NGATHER):
        sl = pl.ds(j * _GCHUNK, _GCHUNK)
        cps.append(
            pltpu.async_copy(tbl_hbm.at[row_v.at[sl]], rows_v.at[sl], sem))
    for cp in cps:
        cp.wait()

    pltpu.sync_copy(rows_v, wide_hbm.at[pl.ds(base, _BPW)])


def _tc_extract_body(col_ref, wide_ref, o_ref):
    col = col_ref[0, 0, :]
    w = wide_ref[...]
    colb = col[:, None]
    acc = jnp.zeros((_TCB, _NUM_ACTIONS), jnp.float32)
    for s in range(_PACK):
        m = colb == (s * _NUM_ACTIONS)
        acc = acc + jnp.where(m, w[:, s * _NUM_ACTIONS:(s + 1) * _NUM_ACTIONS], 0.0)
    o_ref[...] = acc


_tc_extract = pl.pallas_call(
    _tc_extract_body,
    grid=(_TCG,),
    in_specs=[
        pl.BlockSpec((1, 1, _TCB), lambda i: (i, 0, 0)),
        pl.BlockSpec((_TCB, 128), lambda i: (i, 0)),
    ],
    out_specs=pl.BlockSpec((_TCB, _NUM_ACTIONS), lambda i: (i, 0)),
    out_shape=jax.ShapeDtypeStruct((_BATCH, _NUM_ACTIONS), jnp.float32),
)


def kernel(x, params):
    tbl = params.reshape(_PROWS, 128)
    wide, col = _sc_gather(x[0], x[1], x[2], tbl)
    return _tc_extract(col.reshape(_TCG, 1, _TCB), wide)


# speedup vs baseline: 1.0439x; 1.0439x over previous
import functools

import jax
import jax.numpy as jnp
from jax import lax
from jax.experimental import pallas as pl
from jax.experimental.pallas import tpu as pltpu
from jax.experimental.pallas import tpu_sc as plsc

_OBS = (100, 100, 100)
_NUM_ACTIONS = 16
_BATCH = 16384

_NC = 2
_NS = 16
_NW = _NC * _NS
_BPW = _BATCH // _NW
_LANES = 16
_GCHUNK = 128
_NGATHER = _BPW // _GCHUNK


@functools.partial(
    pl.kernel,
    out_type=jax.ShapeDtypeStruct((_BATCH, _NUM_ACTIONS), jnp.float32),
    mesh=plsc.VectorSubcoreMesh(core_axis_name="c", subcore_axis_name="s"),
    scratch_types=[
        pltpu.VMEM((_BPW,), jnp.int32),
        pltpu.VMEM((_BPW,), jnp.int32),
        pltpu.VMEM((_BPW,), jnp.int32),
        pltpu.VMEM((_BPW,), jnp.int32),
        pltpu.VMEM((_BPW, _NUM_ACTIONS), jnp.float32),
        pltpu.SemaphoreType.DMA,
    ],
    compiler_params=pltpu.CompilerParams(use_tc_tiling_on_sc=False),
)
def _sc_gather(x0_hbm, x1_hbm, x2_hbm, params_hbm, out_hbm,
               x0_v, x1_v, x2_v, idx_v, rows_v, sem):
    wid = lax.axis_index("s") * _NC + lax.axis_index("c")
    base = wid * _BPW

    pltpu.sync_copy(x0_hbm.at[pl.ds(base, _BPW)], x0_v)
    pltpu.sync_copy(x1_hbm.at[pl.ds(base, _BPW)], x1_v)
    pltpu.sync_copy(x2_hbm.at[pl.ds(base, _BPW)], x2_v)

    hi = jnp.full((_LANES,), _OBS[0] - 1, jnp.int32)
    lo = jnp.zeros((_LANES,), jnp.int32)
    for i in range(_BPW // _LANES):
        sl = pl.ds(i * _LANES, _LANES)
        a = jnp.minimum(jnp.maximum(x0_v[sl], lo), hi)
        b = jnp.minimum(jnp.maximum(x1_v[sl], lo), hi)
        c = jnp.minimum(jnp.maximum(x2_v[sl], lo), hi)
        idx_v[sl] = (a * (_OBS[1] * _OBS[2]) + b * _OBS[2]) + c

    copies = []
    for j in range(_NGATHER):
        sl = pl.ds(j * _GCHUNK, _GCHUNK)
        copies.append(
            pltpu.async_copy(params_hbm.at[idx_v.at[sl]], rows_v.at[sl], sem))
    for cp in copies:
        cp.wait()

    pltpu.sync_copy(rows_v, out_hbm.at[pl.ds(base, _BPW)])


def kernel(x, params):
    return _sc_gather(x[0], x[1], x[2], params)
